# baseline (speedup 1.0000x reference)
import functools

import jax
import jax.numpy as jnp
from jax import lax
from jax.experimental import pallas as pl
from jax.experimental.pallas import tpu as pltpu
from jax.experimental.pallas import tpu_sc as plsc

F = 26
V = 100000
D = 16
B = 16384

_info = plsc.get_sparse_core_info()
NC, NS, L = _info.num_cores, _info.num_subcores, _info.num_lanes
J = F * D
SPC = J // NC
BPT = B // NS
NBUF = 8
NIV = 4

_mesh = plsc.VectorSubcoreMesh(core_axis_name="c", subcore_axis_name="s")


@functools.partial(
    pl.kernel,
    out_type=jax.ShapeDtypeStruct((J, B), jnp.float32),
    mesh=_mesh,
    compiler_params=pltpu.CompilerParams(use_tc_tiling_on_sc=True),
    scratch_types=[
        [pltpu.VMEM_SHARED((V,), jnp.float32) for _ in range(NBUF)],
        [pltpu.VMEM((BPT,), jnp.int32) for _ in range(NIV)],
        [pltpu.VMEM((BPT,), jnp.float32) for _ in range(NBUF)],
        pltpu.SemaphoreType.DMA,
        pltpu.SemaphoreType.DMA,
        pltpu.SemaphoreType.DMA,
        pltpu.SemaphoreType.DMA,
    ],
)
def _sc_lookup(tab_hbm, idx_hbm, out_hbm, st, iv, cv, fsem, isem, gsem, wsem):
    c = lax.axis_index("c")
    s = lax.axis_index("s")
    j0 = c * SPC
    col = pl.ds(s * BPT, BPT)

    for p in range(3):
        @pl.when(s == p)
        def _prime_fill(p=p):
            pltpu.async_copy(tab_hbm.at[(j0 + p) >> 4, (j0 + p) & 15], st[p], fsem)

    pltpu.sync_copy(idx_hbm.at[j0 >> 4, col], iv[0])

    @pl.when(s == 0)
    def _wait_fill0():
        pltpu.make_async_copy(tab_hbm.at[j0 >> 4, j0 & 15], st[0], fsem).wait()

    plsc.subcore_barrier()

    def octo(t, carry):
        for q in range(NBUF):
            r = NBUF * t + q
            j = j0 + r
            jn = j + 1
            qp = (q - 1) % NBUF
            has_next = r + 1 < SPC
            has_far = r + 3 < SPC

            @pl.when((s == ((r + 3) & 15)) & has_far)
            def _start_fill():
                jf = j + 3
                pltpu.async_copy(tab_hbm.at[jf >> 4, jf & 15], st[(q + 3) % NBUF], fsem)

            @pl.when(has_next)
            def _start_idx():
                pltpu.async_copy(idx_hbm.at[jn >> 4, col], iv[(q + 1) % NIV], isem)

            @pl.when(r >= NBUF)
            def _drain_old_write():
                pltpu.make_async_copy(cv[q], out_hbm.at[j, col], wsem).wait()

            pltpu.async_copy(st[q].at[iv[q % NIV]], cv[q], gsem)

            @pl.when(r >= 1)
            def _finish_prev_gather():
                pltpu.make_async_copy(st[qp].at[iv[qp % NIV]], cv[qp], gsem).wait()
                pltpu.async_copy(cv[qp], out_hbm.at[j - 1, col], wsem)

            @pl.when(has_next)
            def _wait_idx():
                pltpu.make_async_copy(idx_hbm.at[jn >> 4, col], iv[(q + 1) % NIV], isem).wait()

            @pl.when((s == ((r + 1) & 15)) & has_next)
            def _wait_fill():
                pltpu.make_async_copy(tab_hbm.at[jn >> 4, jn & 15], st[(q + 1) % NBUF], fsem).wait()

            plsc.subcore_barrier()
        return carry

    lax.fori_loop(0, SPC // NBUF, octo, 0)
    qlast = (SPC - 1) % NBUF
    pltpu.make_async_copy(st[qlast].at[iv[qlast % NIV]], cv[qlast], gsem).wait()
    pltpu.async_copy(cv[qlast], out_hbm.at[j0 + SPC - 1, col], wsem)
    for q in range(NBUF):
        pltpu.make_async_copy(cv[q], out_hbm.at[j0, col], wsem).wait()


def kernel(indices, tables):
    tab2 = jnp.transpose(tables, (0, 2, 1))
    out = _sc_lookup(tab2, indices)
    return out.T


# speedup vs baseline: 1.3884x; 1.3884x over previous
import functools

import jax
import jax.numpy as jnp
from jax import lax
from jax.experimental import pallas as pl
from jax.experimental.pallas import tpu as pltpu
from jax.experimental.pallas import tpu_sc as plsc

F = 26
V = 100000
D = 16
B = 16384

_info = plsc.get_sparse_core_info()
NC, NS, L = _info.num_cores, _info.num_subcores, _info.num_lanes
J = F * D
SPC = J // NC
BPT = B // NS
NBUF = 8
NIV = 4

_mesh = plsc.VectorSubcoreMesh(core_axis_name="c", subcore_axis_name="s")


@functools.partial(
    pl.kernel,
    out_type=jax.ShapeDtypeStruct((J, B), jnp.float32),
    mesh=_mesh,
    compiler_params=pltpu.CompilerParams(use_tc_tiling_on_sc=True),
    scratch_types=[
        [pltpu.VMEM_SHARED((V,), jnp.float32) for _ in range(NBUF)],
        [pltpu.VMEM((BPT,), jnp.int32) for _ in range(NIV)],
        [pltpu.VMEM((BPT,), jnp.float32) for _ in range(NBUF)],
        pltpu.SemaphoreType.DMA,
        pltpu.SemaphoreType.DMA,
        pltpu.SemaphoreType.DMA,
        pltpu.SemaphoreType.DMA,
    ],
)
def _sc_lookup(tab_hbm, idx_hbm, out_hbm, st, iv, cv, fsem, isem, gsem, wsem):
    c = lax.axis_index("c")
    s = lax.axis_index("s")
    j0 = c * SPC
    col = pl.ds(s * BPT, BPT)

    for p in range(4):
        @pl.when(s == p)
        def _prime_fill(p=p):
            pltpu.async_copy(tab_hbm.at[(j0 + p) >> 4, (j0 + p) & 15], st[p], fsem)

    pltpu.sync_copy(idx_hbm.at[j0 >> 4, col], iv[0])
    pltpu.sync_copy(idx_hbm.at[(j0 + 1) >> 4, col], iv[1])

    for p in range(2):
        @pl.when(s == p)
        def _wait_prime(p=p):
            pltpu.make_async_copy(tab_hbm.at[(j0 + p) >> 4, (j0 + p) & 15], st[p], fsem).wait()

    plsc.subcore_barrier()

    def quad(t, carry):
        for kk in range(4):
            a = 8 * t + 2 * kk
            b = a + 1
            qa, qb = 2 * kk, 2 * kk + 1
            ja, jb = j0 + a, j0 + b

            for off, slot in ((4, (qa + 4) % NBUF), (5, (qb + 4) % NBUF)):
                @pl.when((s == ((a + off) & 15)) & (a + off < SPC))
                def _start_fill(off=off, slot=slot):
                    jf = ja + off
                    pltpu.async_copy(tab_hbm.at[jf >> 4, jf & 15], st[slot], fsem)

            for off in (2, 3):
                @pl.when(a + off < SPC)
                def _start_idx(off=off):
                    jn = ja + off
                    pltpu.async_copy(idx_hbm.at[jn >> 4, col], iv[(2 * kk + off) % NIV], isem)

            @pl.when(a >= NBUF)
            def _drain_old_writes():
                pltpu.make_async_copy(cv[qa], out_hbm.at[ja, col], wsem).wait()
                pltpu.make_async_copy(cv[qb], out_hbm.at[jb, col], wsem).wait()

            ga = pltpu.async_copy(st[qa].at[iv[(2 * kk) % NIV]], cv[qa], gsem)
            gb = pltpu.async_copy(st[qb].at[iv[(2 * kk + 1) % NIV]], cv[qb], gsem)
            ga.wait()
            gb.wait()
            pltpu.async_copy(cv[qa], out_hbm.at[ja, col], wsem)
            pltpu.async_copy(cv[qb], out_hbm.at[jb, col], wsem)

            for off in (2, 3):
                @pl.when(a + off < SPC)
                def _wait_idx(off=off):
                    jn = ja + off
                    pltpu.make_async_copy(idx_hbm.at[jn >> 4, col], iv[(2 * kk + off) % NIV], isem).wait()

            for off in (2, 3):
                @pl.when((s == ((a + off) & 15)) & (a + off < SPC))
                def _wait_fill(off=off, kkoff=off):
                    jn = ja + off
                    pltpu.make_async_copy(tab_hbm.at[jn >> 4, jn & 15], st[(qa + off) % NBUF], fsem).wait()

            plsc.subcore_barrier()
        return carry

    lax.fori_loop(0, SPC // NBUF, quad, 0)
    for q in range(NBUF):
        pltpu.make_async_copy(cv[q], out_hbm.at[j0, col], wsem).wait()


def kernel(indices, tables):
    tab2 = jnp.transpose(tables, (0, 2, 1))
    out = _sc_lookup(tab2, indices)
    return out.T


# speedup vs baseline: 1.6223x; 1.1685x over previous
import functools

import jax
import jax.numpy as jnp
from jax import lax
from jax.experimental import pallas as pl
from jax.experimental.pallas import tpu as pltpu
from jax.experimental.pallas import tpu_sc as plsc

F = 26
V = 100000
D = 16
B = 16384

_info = plsc.get_sparse_core_info()
NC, NS, L = _info.num_cores, _info.num_subcores, _info.num_lanes
J = F * D
SPC = J // NC
BPT = B // NS
G = 4
NBUF = 16
NIV = 8
FD = 8

_mesh = plsc.VectorSubcoreMesh(core_axis_name="c", subcore_axis_name="s")


@functools.partial(
    pl.kernel,
    out_type=jax.ShapeDtypeStruct((J, B), jnp.float32),
    mesh=_mesh,
    compiler_params=pltpu.CompilerParams(use_tc_tiling_on_sc=True),
    scratch_types=[
        [pltpu.VMEM_SHARED((V,), jnp.float32) for _ in range(NBUF)],
        [pltpu.VMEM((BPT,), jnp.int32) for _ in range(NIV)],
        [pltpu.VMEM((BPT,), jnp.float32) for _ in range(NBUF)],
        pltpu.SemaphoreType.DMA,
        pltpu.SemaphoreType.DMA,
        pltpu.SemaphoreType.DMA,
        pltpu.SemaphoreType.DMA,
    ],
)
def _sc_lookup(tab_hbm, idx_hbm, out_hbm, st, iv, cv, fsem, isem, gsem, wsem):
    c = lax.axis_index("c")
    s = lax.axis_index("s")
    j0 = c * SPC
    col = pl.ds(s * BPT, BPT)

    for p in range(FD):
        @pl.when(s == p)
        def _prime_fill(p=p):
            pltpu.async_copy(tab_hbm.at[(j0 + p) >> 4, (j0 + p) & 15], st[p], fsem)

    for p in range(G):
        pltpu.sync_copy(idx_hbm.at[(j0 + p) >> 4, col], iv[p])

    for p in range(G):
        @pl.when(s == p)
        def _wait_prime(p=p):
            pltpu.make_async_copy(tab_hbm.at[(j0 + p) >> 4, (j0 + p) & 15], st[p], fsem).wait()

    plsc.subcore_barrier()

    def body(t, carry):
        for kk in range(NBUF // G):
            a = NBUF * t + G * kk
            q0 = G * kk

            for i in range(G):
                @pl.when((s == ((a + FD + i) & 15)) & (a + FD + i < SPC))
                def _start_fill(i=i):
                    jf = j0 + a + FD + i
                    pltpu.async_copy(tab_hbm.at[jf >> 4, jf & 15],
                                     st[(q0 + FD + i) % NBUF], fsem)

            for i in range(G):
                @pl.when(a + G + i < SPC)
                def _start_idx(i=i):
                    jn = j0 + a + G + i
                    pltpu.async_copy(idx_hbm.at[jn >> 4, col],
                                     iv[(q0 + G + i) % NIV], isem)

            @pl.when(a >= NBUF)
            def _drain_old_writes():
                for i in range(G):
                    pltpu.make_async_copy(cv[q0 + i], out_hbm.at[j0 + a + i, col],
                                          wsem).wait()

            gs = [pltpu.async_copy(st[q0 + i].at[iv[(q0 + i) % NIV]],
                                   cv[q0 + i], gsem) for i in range(G)]
            for g in gs:
                g.wait()
            for i in range(G):
                pltpu.async_copy(cv[q0 + i], out_hbm.at[j0 + a + i, col], wsem)

            for i in range(G):
                @pl.when(a + G + i < SPC)
                def _wait_idx(i=i):
                    jn = j0 + a + G + i
                    pltpu.make_async_copy(idx_hbm.at[jn >> 4, col],
                                          iv[(q0 + G + i) % NIV], isem).wait()

            for i in range(G):
                @pl.when((s == ((a + G + i) & 15)) & (a + G + i < SPC))
                def _wait_fill(i=i):
                    jn = j0 + a + G + i
                    pltpu.make_async_copy(tab_hbm.at[jn >> 4, jn & 15],
                                          st[(q0 + G + i) % NBUF], fsem).wait()

            plsc.subcore_barrier()
        return carry

    lax.fori_loop(0, SPC // NBUF, body, 0)
    for q in range(NBUF):
        pltpu.make_async_copy(cv[q], out_hbm.at[j0, col], wsem).wait()


def kernel(indices, tables):
    tab2 = jnp.transpose(tables, (0, 2, 1))
    out = _sc_lookup(tab2, indices)
    return out.T
